# baseline (speedup 1.0000x reference)
import functools

import jax
import jax.numpy as jnp
from jax import lax
from jax.experimental import pallas as pl
from jax.experimental.pallas import tpu as pltpu
from jax.experimental.pallas import tpu_sc as plsc

_NC = 2
_TOT_U = 1_000_000 * 64
_TOT_I = 100_000 * 64
_PER_U = _TOT_U // _NC
_PER_I = _TOT_I // _NC
_CH = 800_000
_USER_CHUNKS = _PER_U // _CH
_ITEM_CHUNKS = _PER_I // _CH

_mesh = plsc.ScalarSubcoreMesh(axis_name="c", num_cores=_NC)


def _copy_shard(src, dst, base, n_chunks, bufs, in_sems, out_sems):
    def gather(g, b):
        return pltpu.make_async_copy(
            src.at[pl.ds(base + g * _CH, _CH)], bufs[b], in_sems.at[b])

    def scatter(g, b):
        return pltpu.make_async_copy(
            bufs[b], dst.at[pl.ds(base + g * _CH, _CH)], out_sems.at[b])

    gather(0, 0).start()
    for g in range(n_chunks):
        b = g % 2
        gather(g, b).wait()
        scatter(g, b).start()
        if g + 1 < n_chunks:
            if g >= 1:
                scatter(g - 1, 1 - b).wait()
            gather(g + 1, 1 - b).start()
    if n_chunks >= 2:
        scatter(n_chunks - 2, (n_chunks - 2) % 2).wait()
    scatter(n_chunks - 1, (n_chunks - 1) % 2).wait()


@functools.partial(
    pl.kernel,
    out_type=[
        jax.ShapeDtypeStruct((_TOT_U,), jnp.float32),
        jax.ShapeDtypeStruct((_TOT_I,), jnp.float32),
    ],
    mesh=_mesh,
    scratch_types=[
        pltpu.VMEM_SHARED((2 * _CH,), jnp.float32),
        pltpu.SemaphoreType.DMA((2,)),
        pltpu.SemaphoreType.DMA((2,)),
    ],
)
def _sc_copy(u_hbm, i_hbm, out_u, out_i, sbuf, in_sems, out_sems):
    wid = lax.axis_index("c")
    bufs = (sbuf.at[pl.ds(0, _CH)], sbuf.at[pl.ds(_CH, _CH)])
    _copy_shard(u_hbm, out_u, wid * _PER_U, _USER_CHUNKS, bufs, in_sems, out_sems)
    _copy_shard(i_hbm, out_i, wid * _PER_I, _ITEM_CHUNKS, bufs, in_sems, out_sems)


def kernel(emb_user, emb_item):
    out_u, out_i = _sc_copy(emb_user.reshape(_TOT_U), emb_item.reshape(_TOT_I))
    return (out_u.reshape(1_000_000, 64), out_i.reshape(100_000, 64))


# speedup vs baseline: 1.2904x; 1.2904x over previous
import functools

import jax
import jax.numpy as jnp
from jax import lax
from jax.experimental import pallas as pl
from jax.experimental.pallas import tpu as pltpu
from jax.experimental.pallas import tpu_sc as plsc

_NC, _NS = 2, 16
_NW = _NC * _NS
_N_U, _N_I, _D = 1_000_000, 100_000, 64
_CHR = 400
_T_U = _N_U // _CHR
_T_I = _N_I // _CHR
_G_U = -(-_T_U // _NW)
_G_I = -(-_T_I // _NW)

_mesh = plsc.VectorSubcoreMesh(core_axis_name="c", subcore_axis_name="s")


def _copy_shard(src, dst, total, n, wid, bufs, in_sems, out_sems):
    def valid(g):
        return g * _NW + wid < total

    def gather(g, b):
        start = pl.multiple_of((g * _NW + wid) * _CHR, 8)
        return pltpu.make_async_copy(
            src.at[pl.ds(start, _CHR)], bufs[b], in_sems.at[b])

    def scatter(g, b):
        start = pl.multiple_of((g * _NW + wid) * _CHR, 8)
        return pltpu.make_async_copy(
            bufs[b], dst.at[pl.ds(start, _CHR)], out_sems.at[b])

    def do(g, action):
        @pl.when(valid(g))
        def _():
            action()

    do(0, lambda: gather(0, 0).start())
    for g in range(n):
        b = g % 2
        do(g, lambda g=g, b=b: gather(g, b).wait())
        do(g, lambda g=g, b=b: scatter(g, b).start())
        if g + 1 < n:
            if g >= 1:
                do(g - 1, lambda g=g, b=b: scatter(g - 1, 1 - b).wait())
            do(g + 1, lambda g=g, b=b: gather(g + 1, 1 - b).start())
    if n >= 2:
        do(n - 2, lambda: scatter(n - 2, (n - 2) % 2).wait())
    do(n - 1, lambda: scatter(n - 1, (n - 1) % 2).wait())


@functools.partial(
    pl.kernel,
    out_type=[
        jax.ShapeDtypeStruct((_N_U, _D), jnp.float32),
        jax.ShapeDtypeStruct((_N_I, _D), jnp.float32),
    ],
    mesh=_mesh,
    scratch_types=[
        pltpu.VMEM((_CHR, _D), jnp.float32),
        pltpu.VMEM((_CHR, _D), jnp.float32),
        pltpu.SemaphoreType.DMA((2,)),
        pltpu.SemaphoreType.DMA((2,)),
    ],
)
def _sc_copy(u_hbm, i_hbm, out_u, out_i, buf0, buf1, in_sems, out_sems):
    wid = lax.axis_index("s") * _NC + lax.axis_index("c")
    bufs = (buf0, buf1)
    _copy_shard(u_hbm, out_u, _T_U, _G_U, wid, bufs, in_sems, out_sems)
    _copy_shard(i_hbm, out_i, _T_I, _G_I, wid, bufs, in_sems, out_sems)


def kernel(emb_user, emb_item):
    return tuple(_sc_copy(emb_user, emb_item))


# speedup vs baseline: 1.3535x; 1.0489x over previous
import functools

import jax
import jax.numpy as jnp
from jax import lax
from jax.experimental import pallas as pl
from jax.experimental.pallas import tpu as pltpu
from jax.experimental.pallas import tpu_sc as plsc

_NC, _NS = 2, 16
_NW = _NC * _NS
_N_U, _N_I, _D = 1_000_000, 100_000, 64
_CHR = 400
_T_I = _N_I // _CHR
_G_I = -(-_T_I // _NW)

_mesh = plsc.VectorSubcoreMesh(core_axis_name="c", subcore_axis_name="s")


def _copy_shard(src, dst, total, n, wid, bufs, in_sems, out_sems):
    def valid(g):
        return g * _NW + wid < total

    def gather(g, b):
        start = pl.multiple_of((g * _NW + wid) * _CHR, 8)
        return pltpu.make_async_copy(
            src.at[pl.ds(start, _CHR)], bufs[b], in_sems.at[b])

    def scatter(g, b):
        start = pl.multiple_of((g * _NW + wid) * _CHR, 8)
        return pltpu.make_async_copy(
            bufs[b], dst.at[pl.ds(start, _CHR)], out_sems.at[b])

    def do(g, action):
        @pl.when(valid(g))
        def _():
            action()

    do(0, lambda: gather(0, 0).start())
    for g in range(n):
        b = g % 2
        do(g, lambda g=g, b=b: gather(g, b).wait())
        do(g, lambda g=g, b=b: scatter(g, b).start())
        if g + 1 < n:
            if g >= 1:
                do(g - 1, lambda g=g, b=b: scatter(g - 1, 1 - b).wait())
            do(g + 1, lambda g=g, b=b: gather(g + 1, 1 - b).start())
    if n >= 2:
        do(n - 2, lambda: scatter(n - 2, (n - 2) % 2).wait())
    do(n - 1, lambda: scatter(n - 1, (n - 1) % 2).wait())


@functools.partial(
    pl.kernel,
    out_type=jax.ShapeDtypeStruct((_N_I, _D), jnp.float32),
    mesh=_mesh,
    scratch_types=[
        pltpu.VMEM((_CHR, _D), jnp.float32),
        pltpu.VMEM((_CHR, _D), jnp.float32),
        pltpu.SemaphoreType.DMA((2,)),
        pltpu.SemaphoreType.DMA((2,)),
    ],
)
def _sc_item_copy(i_hbm, out_i, buf0, buf1, in_sems, out_sems):
    wid = lax.axis_index("s") * _NC + lax.axis_index("c")
    _copy_shard(i_hbm, out_i, _T_I, _G_I, wid, (buf0, buf1), in_sems, out_sems)


_UB = 8000
_UG = _N_U // _UB


def _tc_body(src, dst):
    dst[...] = src[...]


_tc_user_copy = pl.pallas_call(
    _tc_body,
    grid=(_UG,),
    in_specs=[pl.BlockSpec((_UB, _D), lambda i: (i, 0))],
    out_specs=pl.BlockSpec((_UB, _D), lambda i: (i, 0)),
    out_shape=jax.ShapeDtypeStruct((_N_U, _D), jnp.float32),
)


def kernel(emb_user, emb_item):
    return (_tc_user_copy(emb_user), _sc_item_copy(emb_item))


# speedup vs baseline: 7.7275x; 5.7091x over previous
import functools

import jax
import jax.numpy as jnp
from jax import lax
from jax.experimental import pallas as pl
from jax.experimental.pallas import tpu as pltpu
from jax.experimental.pallas import tpu_sc as plsc

_NC, _NS = 2, 16
_NW = _NC * _NS
_N_U, _N_I, _D = 1_000_000, 100_000, 64

_CW = 640
_T_I = _N_I // _CW
_REM = _N_I - _T_I * _CW
_G_I = -(-_T_I // _NW)

_mesh = plsc.VectorSubcoreMesh(core_axis_name="c", subcore_axis_name="s")


@functools.partial(
    pl.kernel,
    out_type=jax.ShapeDtypeStruct((_D, _N_I), jnp.float32),
    mesh=_mesh,
    scratch_types=[
        pltpu.VMEM((_D, _CW), jnp.float32),
        pltpu.VMEM((_D, _CW), jnp.float32),
        pltpu.SemaphoreType.DMA((2,)),
        pltpu.SemaphoreType.DMA((2,)),
    ],
)
def _sc_item_copy(i_hbm, out_i, buf0, buf1, in_sems, out_sems):
    wid = lax.axis_index("s") * _NC + lax.axis_index("c")
    bufs = (buf0, buf1)

    def valid(g):
        return g * _NW + wid < _T_I

    def gather(g, b):
        start = pl.multiple_of((g * _NW + wid) * _CW, 128)
        return pltpu.make_async_copy(
            i_hbm.at[:, pl.ds(start, _CW)], bufs[b], in_sems.at[b])

    def scatter(g, b):
        start = pl.multiple_of((g * _NW + wid) * _CW, 128)
        return pltpu.make_async_copy(
            bufs[b], out_i.at[:, pl.ds(start, _CW)], out_sems.at[b])

    def do(g, action):
        @pl.when(valid(g))
        def _():
            action()

    n = _G_I
    do(0, lambda: gather(0, 0).start())
    for g in range(n):
        b = g % 2
        do(g, lambda g=g, b=b: gather(g, b).wait())
        do(g, lambda g=g, b=b: scatter(g, b).start())
        if g + 1 < n:
            if g >= 1:
                do(g - 1, lambda g=g, b=b: scatter(g - 1, 1 - b).wait())
            do(g + 1, lambda g=g, b=b: gather(g + 1, 1 - b).start())
    if n >= 2:
        do(n - 2, lambda: scatter(n - 2, (n - 2) % 2).wait())
    do(n - 1, lambda: scatter(n - 1, (n - 1) % 2).wait())


_UB = 16384
_UG = -(-_N_U // _UB)


def _tc_body(src, dst):
    dst[...] = src[...]


_tc_user_copy = pl.pallas_call(
    _tc_body,
    grid=(_UG,),
    in_specs=[pl.BlockSpec((_D, _UB), lambda i: (0, i))],
    out_specs=pl.BlockSpec((_D, _UB), lambda i: (0, i)),
    out_shape=jax.ShapeDtypeStruct((_D, _N_U), jnp.float32),
)


def kernel(emb_user, emb_item):
    u_t, i_t = emb_user.T, emb_item.T
    out_u = _tc_user_copy(u_t)
    out_i = _sc_item_copy(i_t)
    out_i = lax.dynamic_update_slice(out_i, i_t[:, _T_I * _CW:], (0, _T_I * _CW))
    return (out_u.T, out_i.T)


# speedup vs baseline: 7.8639x; 1.0176x over previous
import functools

import jax
import jax.numpy as jnp
from jax import lax
from jax.experimental import pallas as pl
from jax.experimental.pallas import tpu as pltpu
from jax.experimental.pallas import tpu_sc as plsc

_NC, _NS = 2, 16
_NW = _NC * _NS
_N_U, _N_I, _D = 1_000_000, 100_000, 64

_CW = 640
_T_I = _N_I // _CW
_REM = _N_I - _T_I * _CW
_G_I = -(-_T_I // _NW)

_mesh = plsc.VectorSubcoreMesh(core_axis_name="c", subcore_axis_name="s")


@functools.partial(
    pl.kernel,
    out_type=jax.ShapeDtypeStruct((_D, _N_I), jnp.float32),
    mesh=_mesh,
    scratch_types=[
        pltpu.VMEM((_D, _CW), jnp.float32),
        pltpu.VMEM((_D, _CW), jnp.float32),
        pltpu.SemaphoreType.DMA((2,)),
        pltpu.SemaphoreType.DMA((2,)),
    ],
)
def _sc_item_copy(i_hbm, out_i, buf0, buf1, in_sems, out_sems):
    wid = lax.axis_index("s") * _NC + lax.axis_index("c")
    bufs = (buf0, buf1)

    def valid(g):
        return g * _NW + wid < _T_I

    def gather(g, b):
        start = pl.multiple_of((g * _NW + wid) * _CW, 128)
        return pltpu.make_async_copy(
            i_hbm.at[:, pl.ds(start, _CW)], bufs[b], in_sems.at[b])

    def scatter(g, b):
        start = pl.multiple_of((g * _NW + wid) * _CW, 128)
        return pltpu.make_async_copy(
            bufs[b], out_i.at[:, pl.ds(start, _CW)], out_sems.at[b])

    def do(g, action):
        @pl.when(valid(g))
        def _():
            action()

    n = _G_I
    do(0, lambda: gather(0, 0).start())
    for g in range(n):
        b = g % 2
        do(g, lambda g=g, b=b: gather(g, b).wait())
        do(g, lambda g=g, b=b: scatter(g, b).start())
        if g + 1 < n:
            if g >= 1:
                do(g - 1, lambda g=g, b=b: scatter(g - 1, 1 - b).wait())
            do(g + 1, lambda g=g, b=b: gather(g + 1, 1 - b).start())
    if n >= 2:
        do(n - 2, lambda: scatter(n - 2, (n - 2) % 2).wait())
    do(n - 1, lambda: scatter(n - 1, (n - 1) % 2).wait())


_UB = 32768
_UG = -(-_N_U // _UB)


def _tc_body(src, dst):
    dst[...] = src[...]


_tc_user_copy = pl.pallas_call(
    _tc_body,
    grid=(_UG,),
    in_specs=[pl.BlockSpec((_D, _UB), lambda i: (0, i))],
    out_specs=pl.BlockSpec((_D, _UB), lambda i: (0, i)),
    out_shape=jax.ShapeDtypeStruct((_D, _N_U), jnp.float32),
)


def kernel(emb_user, emb_item):
    u_t, i_t = emb_user.T, emb_item.T
    out_u = _tc_user_copy(u_t)
    out_i = _sc_item_copy(i_t)
    out_i = lax.dynamic_update_slice(out_i, i_t[:, _T_I * _CW:], (0, _T_I * _CW))
    return (out_u.T, out_i.T)
